# baseline (speedup 1.0000x reference)
import jax
import jax.numpy as jnp
from jax import lax
from jax.experimental import pallas as pl
from jax.experimental.pallas import tpu as pltpu
from jax.experimental.pallas import tpu_sc as plsc

D = 64
B = 1024
L = 200
LANES = 16

NC = 2
NS = 16
NW = NC * NS
ROWS = B * L
ROWS_PER_W = ROWS // NW
SEQ_PER_W = ROWS_PER_W // L


def _body(ids_hbm, table_hbm, pos_hbm, out_hbm, idx_v, rows_v, pos_v, sem):
    wid = lax.axis_index("s") * NC + lax.axis_index("c")
    base_w = wid * ROWS_PER_W
    pltpu.sync_copy(pos_hbm.at[pl.ds(0, L)], pos_v)

    def seq_body(s, carry):
        base = base_w + s * L
        pltpu.sync_copy(ids_hbm.at[pl.ds(base, L)], idx_v)
        c1 = pltpu.async_copy(
            table_hbm.at[idx_v.at[pl.ds(0, 128)]], rows_v.at[pl.ds(0, 128)], sem)
        c2 = pltpu.async_copy(
            table_hbm.at[idx_v.at[pl.ds(128, L - 128)]],
            rows_v.at[pl.ds(128, L - 128)], sem)
        c1.wait()
        c2.wait()
        for r in range(L):
            for c in range(D // LANES):
                sl = (r, pl.ds(c * LANES, LANES))
                rows_v[sl] = rows_v[sl] + pos_v[sl]
        pltpu.sync_copy(rows_v, out_hbm.at[pl.ds(base, L)])
        return carry

    lax.fori_loop(0, SEQ_PER_W, seq_body, 0)


def kernel(ids, embed_tokens, pos_embed):
    ids_flat = ids.reshape(ROWS).astype(jnp.int32)
    mesh = plsc.VectorSubcoreMesh(core_axis_name="c", subcore_axis_name="s")
    k = pl.kernel(
        _body,
        out_type=jax.ShapeDtypeStruct((ROWS, D), jnp.float32),
        mesh=mesh,
        scratch_types=[
            pltpu.VMEM((L,), jnp.int32),
            pltpu.VMEM((L, D), jnp.float32),
            pltpu.VMEM((L, D), jnp.float32),
            pltpu.SemaphoreType.DMA,
        ],
        compiler_params=pltpu.CompilerParams(use_tc_tiling_on_sc=False),
    )
    out = k(ids_flat, embed_tokens, pos_embed)
    return out.reshape(B, L, D)
